# baseline (speedup 1.0000x reference)
import functools
import math

import jax
import jax.numpy as jnp
from jax import lax
from jax.experimental import pallas as pl
from jax.experimental.pallas import tpu as pltpu
from jax.experimental.pallas import tpu_sc as plsc

EMB = 64
SCALE = math.sqrt(EMB)
NC = 2
NS = 16
NW = NC * NS
CHUNK = 800
LANES = 16


@functools.lru_cache(maxsize=None)
def _make(n_tokens, vocab, interpret=False):
    assert n_tokens % NW == 0
    per_w = n_tokens // NW
    assert per_w % CHUNK == 0
    n_chunks = per_w // CHUNK

    @functools.partial(
        pl.kernel,
        out_type=jax.ShapeDtypeStruct((n_tokens, EMB), jnp.float32),
        mesh=plsc.VectorSubcoreMesh(
            core_axis_name="c", subcore_axis_name="s",
            num_cores=NC, num_subcores=NS,
        ),
        scratch_types=[
            pltpu.VMEM((per_w,), jnp.int32),
            pltpu.VMEM((CHUNK,), jnp.int32),
            pltpu.VMEM((CHUNK, EMB), jnp.float32),
            pltpu.SemaphoreType.DMA,
        ],
        compiler_params=pltpu.CompilerParams(use_tc_tiling_on_sc=False),
        interpret=interpret,
    )
    def emb_kernel(tokens_hbm, table_hbm, out_hbm, idx_v, idx_c, rows_v, gsem):
        wid = lax.axis_index("s") * NC + lax.axis_index("c")
        base = wid * per_w
        pltpu.sync_copy(tokens_hbm.at[pl.ds(base, per_w)], idx_v)

        @pl.loop(0, n_chunks)
        def _chunk(c):
            off = c * CHUNK
            pltpu.async_copy(
                table_hbm.at[idx_v.at[pl.ds(off, CHUNK)]], rows_v, gsem
            ).wait()

            @pl.loop(0, CHUNK, step=4)
            def _scale(i):
                for r in range(4):
                    for j in range(EMB // LANES):
                        sl = (i + r, pl.ds(j * LANES, LANES))
                        rows_v[sl] = rows_v[sl] * SCALE

            pltpu.sync_copy(rows_v, out_hbm.at[pl.ds(base + off, CHUNK)])

    return emb_kernel


def kernel(tokens, embedding):
    b, s = tokens.shape
    flat = tokens.reshape(-1).astype(jnp.int32)
    out = _make(b * s, embedding.shape[0])(flat, embedding)
    return out.reshape(b, s, EMB)
